# baseline (speedup 1.0000x reference)
import functools

import jax
import jax.numpy as jnp
from jax import lax
from jax.experimental import pallas as pl
from jax.experimental.pallas import tpu as pltpu
from jax.experimental.pallas import tpu_sc as plsc

_N = 10000
_E = 320000
_DIN = 128
_DH = 64
_DOUT = 128

_NC = 2
_NS = 16
_K = 128
_EPT = 10240
_E2 = _EPT * _NC * _NS
_CHUNKS = _EPT // _K
_NP = 10112
_RPT = _NP // _NS

_mesh = plsc.VectorSubcoreMesh(core_axis_name="c", subcore_axis_name="s")
_sc_params = pltpu.CompilerParams(use_tc_tiling_on_sc=False)


def _deg_body(dst2_hbm, w_hbm, out_hbm, wtile, dsttile, rows, acc_sh, ssem):
    c = lax.axis_index("c")
    s = lax.axis_index("s")
    i16 = lax.iota(jnp.int32, 16)
    base01 = jnp.where(i16 == 1, 1.0, 0.0).astype(jnp.float32)
    zero16 = jnp.zeros((16,), jnp.float32)

    base = (c * _NS + s) * _EPT
    cbase = (c * _NS + s) * _CHUNKS
    pltpu.sync_copy(w_hbm.at[pl.ds(base, _EPT)], wtile.at[pl.ds(0, _EPT)])
    pltpu.sync_copy(dst2_hbm.at[pl.ds(cbase, _CHUNKS)], dsttile)

    def _zrow(i, carry):
        rows[0, i, :] = zero16
        return carry

    lax.fori_loop(0, _K, _zrow, 0)
    zbase = s * _RPT
    for k in range(_RPT // _K):
        pltpu.sync_copy(rows.at[0], acc_sh.at[pl.ds(zbase + k * _K, _K)])
    if _RPT % _K:
        pltpu.sync_copy(rows.at[0, pl.ds(0, _RPT % _K)],
                        acc_sh.at[pl.ds(zbase + (_RPT // _K) * _K, _RPT % _K)])
    plsc.subcore_barrier()

    def _wait_scat(j, b):
        pltpu.make_async_copy(rows.at[b], acc_sh.at[dsttile.at[j]],
                              ssem.at[b]).wait()

    def _chunk(j, carry):
        b = j % 3

        @pl.when(j >= 3)
        def _():
            _wait_scat(j - 3, b)

        def _build(g):
            v16 = wtile[pl.ds(j * _K + g * 16, 16)]
            for e0 in range(16):
                wv = jnp.full((16,), v16[e0], jnp.float32)
                rows[b, g * 16 + e0, :] = jnp.where(i16 == 0, wv, base01)

        plsc.parallel_loop(0, _K // 16, 1, unroll=2)(_build)
        pltpu.async_copy(rows.at[b], acc_sh.at[dsttile.at[j]], ssem.at[b],
                         add=True)
        return carry

    lax.fori_loop(0, _CHUNKS, _chunk, 0)
    for t in range(3):
        _wait_scat(_CHUNKS - 3 + t, (_CHUNKS - 3 + t) % 3)

    plsc.subcore_barrier()
    pltpu.sync_copy(acc_sh.at[pl.ds(s * _RPT, _RPT)],
                    out_hbm.at[c, pl.ds(s * _RPT, _RPT)])


_deg_call = pl.kernel(
    _deg_body,
    out_type=jax.ShapeDtypeStruct((_NC, _NP, 16), jnp.float32),
    mesh=_mesh,
    compiler_params=_sc_params,
    scratch_types=[
        pltpu.VMEM((_EPT + 16,), jnp.float32),
        pltpu.VMEM((_CHUNKS, _K), jnp.int32),
        pltpu.VMEM((3, _K, 16), jnp.float32),
        pltpu.VMEM_SHARED((_NP, 16), jnp.float32),
        pltpu.SemaphoreType.DMA((3,)),
    ],
)


def _agg_body(weighted, *refs):
    if weighted:
        (src_hbm, dst2_hbm, w_hbm, p_hbm, out_hbm,
         srctile, dsttile, wtile, rows, pbuf_sh, acc_sh, gsem, ssem) = refs
    else:
        (src_hbm, dst2_hbm, p_hbm, out_hbm,
         srctile, dsttile, rows, pbuf_sh, acc_sh, gsem, ssem) = refs
        w_hbm = wtile = None
    c = lax.axis_index("c")
    s = lax.axis_index("s")
    zero16 = jnp.zeros((16,), jnp.float32)

    base = (c * _NS + s) * _EPT
    cbase = (c * _NS + s) * _CHUNKS
    pltpu.sync_copy(p_hbm.at[pl.ds(s * _RPT, _RPT)],
                    pbuf_sh.at[pl.ds(s * _RPT, _RPT)])
    pltpu.sync_copy(src_hbm.at[pl.ds(base, _EPT)], srctile)
    pltpu.sync_copy(dst2_hbm.at[pl.ds(cbase, _CHUNKS)], dsttile)
    if weighted:
        pltpu.sync_copy(w_hbm.at[pl.ds(base, _EPT)], wtile.at[pl.ds(0, _EPT)])

    def _zrow(i, carry):
        for g in range(_DH // 16):
            rows[0, i, pl.ds(g * 16, 16)] = zero16
        return carry

    lax.fori_loop(0, _K, _zrow, 0)
    zbase = s * _RPT
    for k in range(_RPT // _K):
        pltpu.sync_copy(rows.at[0], acc_sh.at[pl.ds(zbase + k * _K, _K)])
    if _RPT % _K:
        pltpu.sync_copy(rows.at[0, pl.ds(0, _RPT % _K)],
                        acc_sh.at[pl.ds(zbase + (_RPT // _K) * _K, _RPT % _K)])
    plsc.subcore_barrier()

    def _issue_gather(j, b):
        pltpu.async_copy(pbuf_sh.at[srctile.at[pl.ds(j * _K, _K)]],
                         rows.at[b], gsem.at[b])

    def _wait_gather(j, b):
        pltpu.make_async_copy(pbuf_sh.at[srctile.at[pl.ds(j * _K, _K)]],
                              rows.at[b], gsem.at[b]).wait()

    def _wait_scat(j, b):
        pltpu.make_async_copy(rows.at[b], acc_sh.at[dsttile.at[j]],
                              ssem.at[b]).wait()

    _issue_gather(0, 0)

    def _chunk(j, carry):
        b = j % 2
        b1 = (j + 1) % 2

        @pl.when(j + 1 < _CHUNKS)
        def _():
            @pl.when(j >= 1)
            def _():
                _wait_scat(j - 1, b1)

            _issue_gather(j + 1, b1)

        _wait_gather(j, b)
        if weighted:

            def _scale(g):
                v16 = wtile[pl.ds(j * _K + g * 16, 16)]
                for e0 in range(16):
                    e = g * 16 + e0
                    wv = jnp.full((16,), v16[e0], jnp.float32)
                    for q in range(_DH // 16):
                        sl = pl.ds(q * 16, 16)
                        rows[b, e, sl] = rows[b, e, sl] * wv

            plsc.parallel_loop(0, _K // 16, 1, unroll=2)(_scale)
        pltpu.async_copy(rows.at[b], acc_sh.at[dsttile.at[j]], ssem.at[b],
                         add=True)
        return carry

    lax.fori_loop(0, _CHUNKS, _chunk, 0)
    _wait_scat(_CHUNKS - 2, (_CHUNKS - 2) % 2)
    _wait_scat(_CHUNKS - 1, (_CHUNKS - 1) % 2)

    plsc.subcore_barrier()
    pltpu.sync_copy(acc_sh.at[pl.ds(s * _RPT, _RPT)],
                    out_hbm.at[c, pl.ds(s * _RPT, _RPT)])


_agg_w_call = pl.kernel(
    functools.partial(_agg_body, True),
    out_type=jax.ShapeDtypeStruct((_NC, _NP, _DH), jnp.float32),
    mesh=_mesh,
    compiler_params=_sc_params,
    scratch_types=[
        pltpu.VMEM((_EPT,), jnp.int32),
        pltpu.VMEM((_CHUNKS, _K), jnp.int32),
        pltpu.VMEM((_EPT + 16,), jnp.float32),
        pltpu.VMEM((2, _K, _DH), jnp.float32),
        pltpu.VMEM_SHARED((_NP, _DH), jnp.float32),
        pltpu.VMEM_SHARED((_NP, _DH), jnp.float32),
        pltpu.SemaphoreType.DMA((2,)),
        pltpu.SemaphoreType.DMA((2,)),
    ],
)

_agg_nw_call = pl.kernel(
    functools.partial(_agg_body, False),
    out_type=jax.ShapeDtypeStruct((_NC, _NP, _DH), jnp.float32),
    mesh=_mesh,
    compiler_params=_sc_params,
    scratch_types=[
        pltpu.VMEM((_EPT,), jnp.int32),
        pltpu.VMEM((_CHUNKS, _K), jnp.int32),
        pltpu.VMEM((2, _K, _DH), jnp.float32),
        pltpu.VMEM_SHARED((_NP, _DH), jnp.float32),
        pltpu.VMEM_SHARED((_NP, _DH), jnp.float32),
        pltpu.SemaphoreType.DMA((2,)),
        pltpu.SemaphoreType.DMA((2,)),
    ],
)


def _tc1a_body(z_ref, w1_ref, h_ref):
    h_ref[...] = jnp.dot(z_ref[...], w1_ref[...],
                         preferred_element_type=jnp.float32)


_tc1a = pl.pallas_call(
    _tc1a_body,
    out_shape=jax.ShapeDtypeStruct((_N, _DH), jnp.float32),
)


def _tc1b_body(deg_ref, h_ref, p1_ref, dinv_ref):
    sl = pl.ds(0, _N)
    d = deg_ref[0, sl, :] + deg_ref[1, sl, :]
    deg1 = d[:, 0:1]
    deg2 = d[:, 1:2]
    dinv1 = jnp.where(deg1 > 0, lax.rsqrt(jnp.where(deg1 > 0, deg1, 1.0)), 0.0)
    dinv2 = jnp.where(deg2 > 0, lax.rsqrt(jnp.where(deg2 > 0, deg2, 1.0)), 0.0)
    p1_ref[sl, :] = h_ref[...] * dinv1
    pad = jnp.zeros_like(dinv1)
    dinv_ref[sl, :] = jnp.concatenate(
        [dinv1, dinv2, pad, pad, pad, pad, pad, pad], axis=1)


_tc1b = pl.pallas_call(
    _tc1b_body,
    out_shape=[
        jax.ShapeDtypeStruct((_NP, _DH), jnp.float32),
        jax.ShapeDtypeStruct((_NP, 8), jnp.float32),
    ],
)


def _tc2_body(acc_ref, dinv_ref, b1_ref, p2_ref):
    a = acc_ref[0] + acc_ref[1]
    d1 = dinv_ref[:, 0:1]
    d2 = dinv_ref[:, 1:2]
    x = jnp.maximum(a * d1 + b1_ref[...], 0.0)
    p2_ref[...] = x * d2


_tc2 = pl.pallas_call(
    _tc2_body,
    out_shape=jax.ShapeDtypeStruct((_NP, _DH), jnp.float32),
)


def _tc3_body(acc_ref, dinv_ref, w2_ref, b2_ref, out_ref):
    a = acc_ref[0] + acc_ref[1]
    d2 = dinv_ref[:, 1:2]
    out_ref[...] = jnp.dot(a * d2, w2_ref[...],
                           preferred_element_type=jnp.float32) + b2_ref[...]


_tc3 = pl.pallas_call(
    _tc3_body,
    grid=(_N // 1000,),
    in_specs=[
        pl.BlockSpec((_NC, 1000, _DH), lambda i: (0, i, 0)),
        pl.BlockSpec((1000, 8), lambda i: (i, 0)),
        pl.BlockSpec((_DH, _DOUT), lambda i: (0, 0)),
        pl.BlockSpec((1, _DOUT), lambda i: (0, 0)),
    ],
    out_specs=pl.BlockSpec((1000, _DOUT), lambda i: (i, 0)),
    out_shape=jax.ShapeDtypeStruct((_N, _DOUT), jnp.float32),
)


def kernel(edge_index, edge_weight, z, W1, b1, W2, b2):
    src = edge_index[0]
    dst = edge_index[1]
    pad = _E2 - _E
    srcp = jnp.concatenate([src, jnp.zeros((pad,), src.dtype)])
    dstp = jnp.concatenate([dst, jnp.full((pad,), _N, dst.dtype)])
    dst2 = dstp.reshape(_E2 // _K, _K)
    wp = jnp.concatenate([edge_weight, jnp.zeros((pad,), edge_weight.dtype)])
    h = _tc1a(z, W1)
    degs = _deg_call(dst2, wp)
    p1, dinv = _tc1b(degs, h)
    acc1 = _agg_w_call(srcp, dst2, wp, p1)
    p2 = _tc2(acc1, dinv, b1.reshape(1, _DH))
    acc2 = _agg_nw_call(srcp, dst2, p2)
    return _tc3(acc2, dinv, W2, b2.reshape(1, _DOUT))


# speedup vs baseline: 1.0574x; 1.0574x over previous
import functools

import jax
import jax.numpy as jnp
from jax import lax
from jax.experimental import pallas as pl
from jax.experimental.pallas import tpu as pltpu
from jax.experimental.pallas import tpu_sc as plsc

_N = 10000
_E = 320000
_DIN = 128
_DH = 64
_DOUT = 128

_NC = 2
_NS = 16
_K = 128
_EPT = 10240
_E2 = _EPT * _NC * _NS
_CHUNKS = _EPT // _K
_NP = 10112
_RPT = _NP // _NS

_mesh = plsc.VectorSubcoreMesh(core_axis_name="c", subcore_axis_name="s")
_sc_params = pltpu.CompilerParams(use_tc_tiling_on_sc=False)


def _deg_body(dst2_hbm, w_hbm, out_hbm, wtile, dsttile, rows, acc_sh, ssem):
    c = lax.axis_index("c")
    s = lax.axis_index("s")
    i16 = lax.iota(jnp.int32, 16)
    base01 = jnp.where(i16 == 1, 1.0, 0.0).astype(jnp.float32)
    zero16 = jnp.zeros((16,), jnp.float32)

    base = (c * _NS + s) * _EPT
    cbase = (c * _NS + s) * _CHUNKS
    pltpu.sync_copy(w_hbm.at[pl.ds(base, _EPT)], wtile.at[pl.ds(0, _EPT)])
    pltpu.sync_copy(dst2_hbm.at[pl.ds(cbase, _CHUNKS)], dsttile)

    def _zrow(i, carry):
        rows[0, i, :] = zero16
        return carry

    lax.fori_loop(0, _K, _zrow, 0)
    zbase = s * _RPT
    for k in range(_RPT // _K):
        pltpu.sync_copy(rows.at[0], acc_sh.at[pl.ds(zbase + k * _K, _K)])
    if _RPT % _K:
        pltpu.sync_copy(rows.at[0, pl.ds(0, _RPT % _K)],
                        acc_sh.at[pl.ds(zbase + (_RPT // _K) * _K, _RPT % _K)])
    plsc.subcore_barrier()

    def _wait_scat(j, b):
        pltpu.make_async_copy(rows.at[b], acc_sh.at[dsttile.at[j]],
                              ssem.at[b]).wait()

    def _chunk(j, carry):
        b = j % 3

        @pl.when(j >= 3)
        def _():
            _wait_scat(j - 3, b)

        def _build(e):
            v = wtile[pl.ds(j * _K + e, 16)]
            wv = jnp.full((16,), v[0], jnp.float32)
            rows[b, e, :] = jnp.where(i16 == 0, wv, base01)

        plsc.parallel_loop(0, _K, 1, unroll=8)(_build)
        pltpu.async_copy(rows.at[b], acc_sh.at[dsttile.at[j]], ssem.at[b],
                         add=True)
        return carry

    lax.fori_loop(0, _CHUNKS, _chunk, 0)
    for t in range(3):
        _wait_scat(_CHUNKS - 3 + t, (_CHUNKS - 3 + t) % 3)

    plsc.subcore_barrier()
    pltpu.sync_copy(acc_sh.at[pl.ds(s * _RPT, _RPT)],
                    out_hbm.at[c, pl.ds(s * _RPT, _RPT)])


_deg_call = pl.kernel(
    _deg_body,
    out_type=jax.ShapeDtypeStruct((_NC, _NP, 16), jnp.float32),
    mesh=_mesh,
    compiler_params=_sc_params,
    scratch_types=[
        pltpu.VMEM((_EPT + 16,), jnp.float32),
        pltpu.VMEM((_CHUNKS, _K), jnp.int32),
        pltpu.VMEM((3, _K, 16), jnp.float32),
        pltpu.VMEM_SHARED((_NP, 16), jnp.float32),
        pltpu.SemaphoreType.DMA((3,)),
    ],
)


def _agg_body(weighted, *refs):
    if weighted:
        (src_hbm, dst2_hbm, w_hbm, p_hbm, out_hbm,
         srctile, dsttile, wtile, rows, pbuf_sh, acc_sh, gsem, ssem) = refs
    else:
        (src_hbm, dst2_hbm, p_hbm, out_hbm,
         srctile, dsttile, rows, pbuf_sh, acc_sh, gsem, ssem) = refs
        w_hbm = wtile = None
    c = lax.axis_index("c")
    s = lax.axis_index("s")
    zero16 = jnp.zeros((16,), jnp.float32)

    base = (c * _NS + s) * _EPT
    cbase = (c * _NS + s) * _CHUNKS
    pltpu.sync_copy(p_hbm.at[pl.ds(s * _RPT, _RPT)],
                    pbuf_sh.at[pl.ds(s * _RPT, _RPT)])
    pltpu.sync_copy(src_hbm.at[pl.ds(base, _EPT)], srctile)
    pltpu.sync_copy(dst2_hbm.at[pl.ds(cbase, _CHUNKS)], dsttile)
    if weighted:
        pltpu.sync_copy(w_hbm.at[pl.ds(base, _EPT)], wtile.at[pl.ds(0, _EPT)])

    def _zrow(i, carry):
        for g in range(_DH // 16):
            rows[0, i, pl.ds(g * 16, 16)] = zero16
        return carry

    lax.fori_loop(0, _K, _zrow, 0)
    zbase = s * _RPT
    for k in range(_RPT // _K):
        pltpu.sync_copy(rows.at[0], acc_sh.at[pl.ds(zbase + k * _K, _K)])
    if _RPT % _K:
        pltpu.sync_copy(rows.at[0, pl.ds(0, _RPT % _K)],
                        acc_sh.at[pl.ds(zbase + (_RPT // _K) * _K, _RPT % _K)])
    plsc.subcore_barrier()

    def _issue_gather(j, b):
        pltpu.async_copy(pbuf_sh.at[srctile.at[pl.ds(j * _K, _K)]],
                         rows.at[b], gsem.at[b])

    def _wait_gather(j, b):
        pltpu.make_async_copy(pbuf_sh.at[srctile.at[pl.ds(j * _K, _K)]],
                              rows.at[b], gsem.at[b]).wait()

    def _wait_scat(j, b):
        pltpu.make_async_copy(rows.at[b], acc_sh.at[dsttile.at[j]],
                              ssem.at[b]).wait()

    _issue_gather(0, 0)

    def _chunk(j, carry):
        b = j % 2
        b1 = (j + 1) % 2

        @pl.when(j + 1 < _CHUNKS)
        def _():
            @pl.when(j >= 1)
            def _():
                _wait_scat(j - 1, b1)

            _issue_gather(j + 1, b1)

        _wait_gather(j, b)
        if weighted:

            def _scale(e):
                v = wtile[pl.ds(j * _K + e, 16)]
                wv = jnp.full((16,), v[0], jnp.float32)
                for q in range(_DH // 16):
                    sl = pl.ds(q * 16, 16)
                    rows[b, e, sl] = rows[b, e, sl] * wv

            plsc.parallel_loop(0, _K, 1, unroll=8)(_scale)
        pltpu.async_copy(rows.at[b], acc_sh.at[dsttile.at[j]], ssem.at[b],
                         add=True)
        return carry

    lax.fori_loop(0, _CHUNKS, _chunk, 0)
    _wait_scat(_CHUNKS - 2, (_CHUNKS - 2) % 2)
    _wait_scat(_CHUNKS - 1, (_CHUNKS - 1) % 2)

    plsc.subcore_barrier()
    pltpu.sync_copy(acc_sh.at[pl.ds(s * _RPT, _RPT)],
                    out_hbm.at[c, pl.ds(s * _RPT, _RPT)])


_agg_w_call = pl.kernel(
    functools.partial(_agg_body, True),
    out_type=jax.ShapeDtypeStruct((_NC, _NP, _DH), jnp.float32),
    mesh=_mesh,
    compiler_params=_sc_params,
    scratch_types=[
        pltpu.VMEM((_EPT,), jnp.int32),
        pltpu.VMEM((_CHUNKS, _K), jnp.int32),
        pltpu.VMEM((_EPT + 16,), jnp.float32),
        pltpu.VMEM((2, _K, _DH), jnp.float32),
        pltpu.VMEM_SHARED((_NP, _DH), jnp.float32),
        pltpu.VMEM_SHARED((_NP, _DH), jnp.float32),
        pltpu.SemaphoreType.DMA((2,)),
        pltpu.SemaphoreType.DMA((2,)),
    ],
)

_agg_nw_call = pl.kernel(
    functools.partial(_agg_body, False),
    out_type=jax.ShapeDtypeStruct((_NC, _NP, _DH), jnp.float32),
    mesh=_mesh,
    compiler_params=_sc_params,
    scratch_types=[
        pltpu.VMEM((_EPT,), jnp.int32),
        pltpu.VMEM((_CHUNKS, _K), jnp.int32),
        pltpu.VMEM((2, _K, _DH), jnp.float32),
        pltpu.VMEM_SHARED((_NP, _DH), jnp.float32),
        pltpu.VMEM_SHARED((_NP, _DH), jnp.float32),
        pltpu.SemaphoreType.DMA((2,)),
        pltpu.SemaphoreType.DMA((2,)),
    ],
)


def _tc1a_body(z_ref, w1_ref, h_ref):
    h_ref[...] = jnp.dot(z_ref[...], w1_ref[...],
                         preferred_element_type=jnp.float32)


_tc1a = pl.pallas_call(
    _tc1a_body,
    out_shape=jax.ShapeDtypeStruct((_N, _DH), jnp.float32),
)


def _tc1b_body(deg_ref, h_ref, p1_ref, dinv_ref):
    sl = pl.ds(0, _N)
    d = deg_ref[0, sl, :] + deg_ref[1, sl, :]
    deg1 = d[:, 0:1]
    deg2 = d[:, 1:2]
    dinv1 = jnp.where(deg1 > 0, lax.rsqrt(jnp.where(deg1 > 0, deg1, 1.0)), 0.0)
    dinv2 = jnp.where(deg2 > 0, lax.rsqrt(jnp.where(deg2 > 0, deg2, 1.0)), 0.0)
    p1_ref[sl, :] = h_ref[...] * dinv1
    pad = jnp.zeros_like(dinv1)
    dinv_ref[sl, :] = jnp.concatenate(
        [dinv1, dinv2, pad, pad, pad, pad, pad, pad], axis=1)


_tc1b = pl.pallas_call(
    _tc1b_body,
    out_shape=[
        jax.ShapeDtypeStruct((_NP, _DH), jnp.float32),
        jax.ShapeDtypeStruct((_NP, 8), jnp.float32),
    ],
)


def _tc2_body(acc_ref, dinv_ref, b1_ref, p2_ref):
    a = acc_ref[0] + acc_ref[1]
    d1 = dinv_ref[:, 0:1]
    d2 = dinv_ref[:, 1:2]
    x = jnp.maximum(a * d1 + b1_ref[...], 0.0)
    p2_ref[...] = x * d2


_tc2 = pl.pallas_call(
    _tc2_body,
    out_shape=jax.ShapeDtypeStruct((_NP, _DH), jnp.float32),
)


def _tc3_body(acc_ref, dinv_ref, w2_ref, b2_ref, out_ref):
    a = acc_ref[0] + acc_ref[1]
    d2 = dinv_ref[:, 1:2]
    out_ref[...] = jnp.dot(a * d2, w2_ref[...],
                           preferred_element_type=jnp.float32) + b2_ref[...]


_tc3 = pl.pallas_call(
    _tc3_body,
    grid=(_N // 1000,),
    in_specs=[
        pl.BlockSpec((_NC, 1000, _DH), lambda i: (0, i, 0)),
        pl.BlockSpec((1000, 8), lambda i: (i, 0)),
        pl.BlockSpec((_DH, _DOUT), lambda i: (0, 0)),
        pl.BlockSpec((1, _DOUT), lambda i: (0, 0)),
    ],
    out_specs=pl.BlockSpec((1000, _DOUT), lambda i: (i, 0)),
    out_shape=jax.ShapeDtypeStruct((_N, _DOUT), jnp.float32),
)


def kernel(edge_index, edge_weight, z, W1, b1, W2, b2):
    src = edge_index[0]
    dst = edge_index[1]
    pad = _E2 - _E
    srcp = jnp.concatenate([src, jnp.zeros((pad,), src.dtype)])
    dstp = jnp.concatenate([dst, jnp.full((pad,), _N, dst.dtype)])
    dst2 = dstp.reshape(_E2 // _K, _K)
    wp = jnp.concatenate([edge_weight, jnp.zeros((pad,), edge_weight.dtype)])
    h = _tc1a(z, W1)
    degs = _deg_call(dst2, wp)
    p1, dinv = _tc1b(degs, h)
    acc1 = _agg_w_call(srcp, dst2, wp, p1)
    p2 = _tc2(acc1, dinv, b1.reshape(1, _DH))
    acc2 = _agg_nw_call(srcp, dst2, p2)
    return _tc3(acc2, dinv, W2, b2.reshape(1, _DOUT))


# speedup vs baseline: 1.2344x; 1.1674x over previous
import functools

import jax
import jax.numpy as jnp
from jax import lax
from jax.experimental import pallas as pl
from jax.experimental.pallas import tpu as pltpu
from jax.experimental.pallas import tpu_sc as plsc

_N = 10000
_E = 320000
_DIN = 128
_DH = 64
_DOUT = 128

_NC = 2
_NS = 16
_K = 128
_EPT = 10240
_CHUNKS = _EPT // _K
_LASTW = _NC * _NS - 1
_ELAST = _E - _LASTW * _EPT
_CHLAST = _ELAST // _K
_NP = 10112
_RPT = _NP // _NS

_mesh = plsc.VectorSubcoreMesh(core_axis_name="c", subcore_axis_name="s")
_sc_params = pltpu.CompilerParams(use_tc_tiling_on_sc=False)


def _deg_body(ei_hbm, w_hbm, out_hbm, wtile, dsttile, rows, acc_sh,
              ssem, dsem):
    c = lax.axis_index("c")
    s = lax.axis_index("s")
    i16 = lax.iota(jnp.int32, 16)
    base01 = jnp.where(i16 == 1, 1.0, 0.0).astype(jnp.float32)
    zero16 = jnp.zeros((16,), jnp.float32)

    wid = c * _NS + s
    base = wid * _EPT
    nch = jnp.where(wid < _LASTW, _CHUNKS, _CHLAST)

    @pl.when(wid < _LASTW)
    def _():
        pltpu.sync_copy(w_hbm.at[pl.ds(base, _EPT)], wtile.at[pl.ds(0, _EPT)])

    @pl.when(wid == _LASTW)
    def _():
        pltpu.sync_copy(w_hbm.at[pl.ds(base, _ELAST)],
                        wtile.at[pl.ds(0, _ELAST)])

    def _issue_dst(j):
        pltpu.async_copy(ei_hbm.at[1, pl.ds(base + j * _K, _K)],
                         dsttile.at[j], dsem.at[j % 4])

    def _wait_dst(j):
        pltpu.make_async_copy(ei_hbm.at[1, pl.ds(base + j * _K, _K)],
                              dsttile.at[j], dsem.at[j % 4]).wait()

    for t in range(3):
        _issue_dst(t)

    def _zrow(i, carry):
        rows[0, i, :] = zero16
        return carry

    lax.fori_loop(0, _K, _zrow, 0)
    zbase = s * _RPT
    for k in range(_RPT // _K):
        pltpu.sync_copy(rows.at[0], acc_sh.at[pl.ds(zbase + k * _K, _K)])
    if _RPT % _K:
        pltpu.sync_copy(rows.at[0, pl.ds(0, _RPT % _K)],
                        acc_sh.at[pl.ds(zbase + (_RPT // _K) * _K, _RPT % _K)])
    plsc.subcore_barrier()

    def _wait_scat(j, b):
        pltpu.make_async_copy(rows.at[b], acc_sh.at[dsttile.at[j]],
                              ssem.at[b]).wait()

    def _chunk(j, carry):
        b = j % 3

        @pl.when(j >= 3)
        def _():
            _wait_scat(j - 3, b)

        @pl.when(j + 3 < nch)
        def _():
            _issue_dst(j + 3)

        def _build(e):
            v = wtile[pl.ds(j * _K + e, 16)]
            wv = jnp.full((16,), v[0], jnp.float32)
            rows[b, e, :] = jnp.where(i16 == 0, wv, base01)

        plsc.parallel_loop(0, _K, 1, unroll=8)(_build)
        _wait_dst(j)
        pltpu.async_copy(rows.at[b], acc_sh.at[dsttile.at[j]], ssem.at[b],
                         add=True)
        return carry

    lax.fori_loop(0, nch, _chunk, 0)
    for t in range(3):
        _wait_scat(nch - 3 + t, (t + 2) % 3)

    plsc.subcore_barrier()
    pltpu.sync_copy(acc_sh.at[pl.ds(s * _RPT, _RPT)],
                    out_hbm.at[c, pl.ds(s * _RPT, _RPT)])


_deg_call = pl.kernel(
    _deg_body,
    out_type=jax.ShapeDtypeStruct((_NC, _NP, 16), jnp.float32),
    mesh=_mesh,
    compiler_params=_sc_params,
    scratch_types=[
        pltpu.VMEM((_EPT + 16,), jnp.float32),
        pltpu.VMEM((_CHUNKS, _K), jnp.int32),
        pltpu.VMEM((3, _K, 16), jnp.float32),
        pltpu.VMEM_SHARED((_NP, 16), jnp.float32),
        pltpu.SemaphoreType.DMA((3,)),
        pltpu.SemaphoreType.DMA((4,)),
    ],
)


def _agg_body(weighted, *refs):
    if weighted:
        (ei_hbm, w_hbm, p_hbm, out_hbm,
         srctile, dsttile, wtile, rows, pbuf_sh, acc_sh,
         gsem, ssem, dsem) = refs
    else:
        (ei_hbm, p_hbm, out_hbm,
         srctile, dsttile, rows, pbuf_sh, acc_sh, gsem, ssem, dsem) = refs
        w_hbm = wtile = None
    c = lax.axis_index("c")
    s = lax.axis_index("s")
    zero16 = jnp.zeros((16,), jnp.float32)

    wid = c * _NS + s
    base = wid * _EPT
    nch = jnp.where(wid < _LASTW, _CHUNKS, _CHLAST)
    pltpu.sync_copy(p_hbm.at[pl.ds(s * _RPT, _RPT)],
                    pbuf_sh.at[pl.ds(s * _RPT, _RPT)])

    @pl.when(wid < _LASTW)
    def _():
        pltpu.sync_copy(ei_hbm.at[0, pl.ds(base, _EPT)], srctile)
        if weighted:
            pltpu.sync_copy(w_hbm.at[pl.ds(base, _EPT)],
                            wtile.at[pl.ds(0, _EPT)])

    @pl.when(wid == _LASTW)
    def _():
        pltpu.sync_copy(ei_hbm.at[0, pl.ds(base, _ELAST)],
                        srctile.at[pl.ds(0, _ELAST)])
        if weighted:
            pltpu.sync_copy(w_hbm.at[pl.ds(base, _ELAST)],
                            wtile.at[pl.ds(0, _ELAST)])

    def _issue_dst(j):
        pltpu.async_copy(ei_hbm.at[1, pl.ds(base + j * _K, _K)],
                         dsttile.at[j], dsem.at[j % 4])

    def _wait_dst(j):
        pltpu.make_async_copy(ei_hbm.at[1, pl.ds(base + j * _K, _K)],
                              dsttile.at[j], dsem.at[j % 4]).wait()

    for t in range(3):
        _issue_dst(t)

    def _zrow(i, carry):
        for g in range(_DH // 16):
            rows[0, i, pl.ds(g * 16, 16)] = zero16
        return carry

    lax.fori_loop(0, _K, _zrow, 0)
    zbase = s * _RPT
    for k in range(_RPT // _K):
        pltpu.sync_copy(rows.at[0], acc_sh.at[pl.ds(zbase + k * _K, _K)])
    if _RPT % _K:
        pltpu.sync_copy(rows.at[0, pl.ds(0, _RPT % _K)],
                        acc_sh.at[pl.ds(zbase + (_RPT // _K) * _K, _RPT % _K)])
    plsc.subcore_barrier()

    def _issue_gather(j, b):
        pltpu.async_copy(pbuf_sh.at[srctile.at[pl.ds(j * _K, _K)]],
                         rows.at[b], gsem.at[b])

    def _wait_gather(j, b):
        pltpu.make_async_copy(pbuf_sh.at[srctile.at[pl.ds(j * _K, _K)]],
                              rows.at[b], gsem.at[b]).wait()

    def _wait_scat(j, b):
        pltpu.make_async_copy(rows.at[b], acc_sh.at[dsttile.at[j]],
                              ssem.at[b]).wait()

    _issue_gather(0, 0)

    def _chunk(j, carry):
        b = j % 2
        b1 = (j + 1) % 2

        @pl.when(j + 1 < nch)
        def _():
            @pl.when(j >= 1)
            def _():
                _wait_scat(j - 1, b1)

            _issue_gather(j + 1, b1)

        @pl.when(j + 3 < nch)
        def _():
            _issue_dst(j + 3)

        _wait_gather(j, b)
        if weighted:

            def _scale(e):
                v = wtile[pl.ds(j * _K + e, 16)]
                wv = jnp.full((16,), v[0], jnp.float32)
                for q in range(_DH // 16):
                    sl = pl.ds(q * 16, 16)
                    rows[b, e, sl] = rows[b, e, sl] * wv

            plsc.parallel_loop(0, _K, 1, unroll=8)(_scale)
        _wait_dst(j)
        pltpu.async_copy(rows.at[b], acc_sh.at[dsttile.at[j]], ssem.at[b],
                         add=True)
        return carry

    lax.fori_loop(0, nch, _chunk, 0)
    _wait_scat(nch - 2, 0)
    _wait_scat(nch - 1, 1)

    plsc.subcore_barrier()
    pltpu.sync_copy(acc_sh.at[pl.ds(s * _RPT, _RPT)],
                    out_hbm.at[c, pl.ds(s * _RPT, _RPT)])


_agg_w_call = pl.kernel(
    functools.partial(_agg_body, True),
    out_type=jax.ShapeDtypeStruct((_NC, _NP, _DH), jnp.float32),
    mesh=_mesh,
    compiler_params=_sc_params,
    scratch_types=[
        pltpu.VMEM((_EPT,), jnp.int32),
        pltpu.VMEM((_CHUNKS, _K), jnp.int32),
        pltpu.VMEM((_EPT + 16,), jnp.float32),
        pltpu.VMEM((2, _K, _DH), jnp.float32),
        pltpu.VMEM_SHARED((_NP, _DH), jnp.float32),
        pltpu.VMEM_SHARED((_NP, _DH), jnp.float32),
        pltpu.SemaphoreType.DMA((2,)),
        pltpu.SemaphoreType.DMA((2,)),
        pltpu.SemaphoreType.DMA((4,)),
    ],
)

_agg_nw_call = pl.kernel(
    functools.partial(_agg_body, False),
    out_type=jax.ShapeDtypeStruct((_NC, _NP, _DH), jnp.float32),
    mesh=_mesh,
    compiler_params=_sc_params,
    scratch_types=[
        pltpu.VMEM((_EPT,), jnp.int32),
        pltpu.VMEM((_CHUNKS, _K), jnp.int32),
        pltpu.VMEM((2, _K, _DH), jnp.float32),
        pltpu.VMEM_SHARED((_NP, _DH), jnp.float32),
        pltpu.VMEM_SHARED((_NP, _DH), jnp.float32),
        pltpu.SemaphoreType.DMA((2,)),
        pltpu.SemaphoreType.DMA((2,)),
        pltpu.SemaphoreType.DMA((4,)),
    ],
)


def _tc1a_body(z_ref, w1_ref, h_ref):
    h_ref[...] = jnp.dot(z_ref[...], w1_ref[...],
                         preferred_element_type=jnp.float32)


_tc1a = pl.pallas_call(
    _tc1a_body,
    out_shape=jax.ShapeDtypeStruct((_N, _DH), jnp.float32),
)


def _tc1b_body(deg_ref, h_ref, p1_ref, dinv_ref):
    sl = pl.ds(0, _N)
    d = deg_ref[0, sl, :] + deg_ref[1, sl, :]
    deg1 = d[:, 0:1]
    deg2 = d[:, 1:2]
    dinv1 = jnp.where(deg1 > 0, lax.rsqrt(jnp.where(deg1 > 0, deg1, 1.0)), 0.0)
    dinv2 = jnp.where(deg2 > 0, lax.rsqrt(jnp.where(deg2 > 0, deg2, 1.0)), 0.0)
    p1_ref[sl, :] = h_ref[...] * dinv1
    pad = jnp.zeros_like(dinv1)
    dinv_ref[sl, :] = jnp.concatenate(
        [dinv1, dinv2, pad, pad, pad, pad, pad, pad], axis=1)


_tc1b = pl.pallas_call(
    _tc1b_body,
    out_shape=[
        jax.ShapeDtypeStruct((_NP, _DH), jnp.float32),
        jax.ShapeDtypeStruct((_NP, 8), jnp.float32),
    ],
)


def _tc2_body(acc_ref, dinv_ref, b1_ref, p2_ref):
    a = acc_ref[0] + acc_ref[1]
    d1 = dinv_ref[:, 0:1]
    d2 = dinv_ref[:, 1:2]
    x = jnp.maximum(a * d1 + b1_ref[...], 0.0)
    p2_ref[...] = x * d2


_tc2 = pl.pallas_call(
    _tc2_body,
    out_shape=jax.ShapeDtypeStruct((_NP, _DH), jnp.float32),
)


def _tc3_body(acc_ref, dinv_ref, w2_ref, b2_ref, out_ref):
    a = acc_ref[0] + acc_ref[1]
    d2 = dinv_ref[:, 1:2]
    out_ref[...] = jnp.dot(a * d2, w2_ref[...],
                           preferred_element_type=jnp.float32) + b2_ref[...]


_tc3 = pl.pallas_call(
    _tc3_body,
    grid=(_N // 1000,),
    in_specs=[
        pl.BlockSpec((_NC, 1000, _DH), lambda i: (0, i, 0)),
        pl.BlockSpec((1000, 8), lambda i: (i, 0)),
        pl.BlockSpec((_DH, _DOUT), lambda i: (0, 0)),
        pl.BlockSpec((1, _DOUT), lambda i: (0, 0)),
    ],
    out_specs=pl.BlockSpec((1000, _DOUT), lambda i: (i, 0)),
    out_shape=jax.ShapeDtypeStruct((_N, _DOUT), jnp.float32),
)


def kernel(edge_index, edge_weight, z, W1, b1, W2, b2):
    h = _tc1a(z, W1)
    degs = _deg_call(edge_index, edge_weight)
    p1, dinv = _tc1b(degs, h)
    acc1 = _agg_w_call(edge_index, edge_weight, p1)
    p2 = _tc2(acc1, dinv, b1.reshape(1, _DH))
    acc2 = _agg_nw_call(edge_index, p2)
    return _tc3(acc2, dinv, W2, b2.reshape(1, _DOUT))
